# initial kernel scaffold (unmeasured)
import functools

import jax
import jax.numpy as jnp
from jax import lax
from jax.experimental import pallas as pl
from jax.experimental.pallas import tpu as pltpu

N_DEV = 4
SQ = 2048
SKV = 2048
H_LOC = 8
DH = 128
D_MODEL = 1024
D_LOC = H_LOC * DH
SCALE = 0.08838834764831843
Q_TILE = 512
N_QT = SQ // Q_TILE


def _body(x_ref, wq_ref, k_ref, v_ref, wo_ref, out_ref,
          q_ref, ctx_ref, comm_ref, send_sems, recv_sems):
    my_pos = lax.axis_index("i")
    left = (my_pos - 1) % N_DEV
    right = (my_pos + 1) % N_DEV

    barrier_sem = pltpu.get_barrier_semaphore()
    for nbr in [left, right]:
        pl.semaphore_signal(
            barrier_sem, inc=1,
            device_id=(nbr,), device_id_type=pl.DeviceIdType.MESH,
        )
    pl.semaphore_wait(barrier_sem, 2)

    q_ref[...] = jnp.dot(
        x_ref[...], wq_ref[...], preferred_element_type=jnp.float32
    ).astype(jnp.bfloat16)

    for h in range(H_LOC):
        kh = k_ref[h]
        vh = v_ref[h]
        for t in range(N_QT):
            r0 = t * Q_TILE
            qt = q_ref[r0:r0 + Q_TILE, h * DH:(h + 1) * DH]
            scores = lax.dot_general(
                qt, kh, (((1,), (1,)), ((), ())),
                preferred_element_type=jnp.float32,
            ) * SCALE
            qb = (r0 + lax.broadcasted_iota(jnp.int32, (Q_TILE, SKV), 0)) // 64
            kb = lax.broadcasted_iota(jnp.int32, (Q_TILE, SKV), 1) // 64
            mask = (qb == kb) | (kb == 0) | ((qb + kb) % 3 == 0)
            scores = jnp.where(mask, scores, -1e9)
            m = jnp.max(scores, axis=1, keepdims=True)
            e = jnp.exp(scores - m)
            s = jnp.sum(e, axis=1, keepdims=True)
            w = (e / s).astype(jnp.bfloat16)
            ctx_ref[r0:r0 + Q_TILE, h * DH:(h + 1) * DH] = jnp.dot(
                w, vh, preferred_element_type=jnp.float32
            ).astype(jnp.bfloat16)

    partial = jnp.dot(
        ctx_ref[...], wo_ref[...], preferred_element_type=jnp.float32
    )
    out_ref[...] = partial
    comm_ref[0, :, :] = partial.astype(jnp.bfloat16)

    for hop in range(N_DEV - 1):
        rdma = pltpu.make_async_remote_copy(
            src_ref=comm_ref.at[hop],
            dst_ref=comm_ref.at[hop + 1],
            send_sem=send_sems.at[hop],
            recv_sem=recv_sems.at[hop],
            device_id=(right,),
            device_id_type=pl.DeviceIdType.MESH,
        )
        rdma.start()
        rdma.wait()
        out_ref[...] += comm_ref[hop + 1].astype(jnp.float32)


def kernel(x, Wq, K_ext, V_ext, Wo):
    my_pos = lax.axis_index("i")
    wq_loc = lax.dynamic_slice(
        Wq, (0, my_pos * D_LOC), (Wq.shape[0], D_LOC)
    ).astype(jnp.bfloat16)
    wo_loc = lax.dynamic_slice(
        Wo, (my_pos * D_LOC, 0), (D_LOC, Wo.shape[1])
    ).astype(jnp.bfloat16)
    xs = x[0].astype(jnp.bfloat16)
    k = K_ext[0].transpose(1, 0, 2).astype(jnp.bfloat16)
    v = V_ext[0].transpose(1, 0, 2).astype(jnp.bfloat16)

    out = pl.pallas_call(
        _body,
        out_shape=jax.ShapeDtypeStruct((SQ, D_MODEL), jnp.float32),
        in_specs=[pl.BlockSpec(memory_space=pltpu.VMEM)] * 5,
        out_specs=pl.BlockSpec(memory_space=pltpu.VMEM),
        scratch_shapes=[
            pltpu.VMEM((SQ, D_LOC), jnp.bfloat16),
            pltpu.VMEM((SQ, D_LOC), jnp.bfloat16),
            pltpu.VMEM((N_DEV, SQ, D_MODEL), jnp.bfloat16),
            pltpu.SemaphoreType.DMA((N_DEV - 1,)),
            pltpu.SemaphoreType.DMA((N_DEV - 1,)),
        ],
        compiler_params=pltpu.CompilerParams(collective_id=0),
    )(xs, wq_loc, k, v, wo_loc)
    return out.reshape(1, SQ, D_MODEL)


# baseline (device time: 262588 ns/iter reference)
import jax
import jax.numpy as jnp
from jax import lax
from jax.experimental import pallas as pl
from jax.experimental.pallas import tpu as pltpu

N_DEV = 4
SQ = 2048
SKV = 2048
H_LOC = 8
DH = 128
D_MODEL = 1024
D_LOC = H_LOC * DH
SCALE = 0.08838834764831843
Q_TILE = 512
N_QT = SQ // Q_TILE
CHUNK = SQ // N_DEV


def _body(x_ref, wq_ref, k_ref, v_ref, wo_ref, out_ref,
          qh_ref, ctx_ref, comm_ref, send_sems, recv_sems):
    my_pos = lax.axis_index("i")
    left = (my_pos - 1) % N_DEV
    right = (my_pos + 1) % N_DEV

    barrier_sem = pltpu.get_barrier_semaphore()
    for nbr in [left, right]:
        pl.semaphore_signal(
            barrier_sem, inc=1,
            device_id=(nbr,), device_id_type=pl.DeviceIdType.MESH,
        )
    pl.semaphore_wait(barrier_sem, 2)

    out_ref[...] = jnp.zeros((SQ, D_MODEL), jnp.float32)

    def head_step(h, _):
        qh_ref[...] = jnp.dot(
            x_ref[...], wq_ref[h], preferred_element_type=jnp.float32
        ).astype(jnp.bfloat16)
        kh = k_ref[h]
        vh = v_ref[h]
        for t in range(N_QT):
            r0 = t * Q_TILE
            scores = lax.dot_general(
                qh_ref[r0:r0 + Q_TILE, :], kh, (((1,), (1,)), ((), ())),
                preferred_element_type=jnp.float32,
            ) * SCALE
            qb = (r0 + lax.broadcasted_iota(jnp.int32, (Q_TILE, SKV), 0)) // 64
            kb = lax.broadcasted_iota(jnp.int32, (Q_TILE, SKV), 1) // 64
            mask = (qb == kb) | (kb == 0) | ((qb + kb) % 3 == 0)
            scores = jnp.where(mask, scores, -1e9)
            m = jnp.max(scores, axis=1, keepdims=True)
            e = jnp.exp(scores - m)
            s = jnp.sum(e, axis=1, keepdims=True)
            w = (e / s).astype(jnp.bfloat16)
            ctx_ref[r0:r0 + Q_TILE, :] = jnp.dot(
                w, vh, preferred_element_type=jnp.float32
            ).astype(jnp.bfloat16)
        out_ref[...] += jnp.dot(
            ctx_ref[...], wo_ref[h], preferred_element_type=jnp.float32
        )
        return 0

    lax.fori_loop(0, H_LOC, head_step, 0)


    def chunk_rows(c):
        return pl.ds(c * CHUNK, CHUNK)

    for s in range(N_DEV - 1):
        cs = (my_pos - s) % N_DEV
        cr = (my_pos - s - 1) % N_DEV
        comm_ref[s] = out_ref[chunk_rows(cs), :].astype(jnp.bfloat16)
        rdma = pltpu.make_async_remote_copy(
            src_ref=comm_ref.at[s],
            dst_ref=comm_ref.at[3 + s],
            send_sem=send_sems.at[s],
            recv_sem=recv_sems.at[s],
            device_id=(right,),
            device_id_type=pl.DeviceIdType.MESH,
        )
        rdma.start()
        rdma.wait()
        out_ref[chunk_rows(cr), :] += comm_ref[3 + s].astype(jnp.float32)

    comm_ref[6] = out_ref[chunk_rows((my_pos + 1) % N_DEV), :].astype(
        jnp.bfloat16
    )
    for t in range(N_DEV - 1):
        gr = (my_pos - t) % N_DEV
        rdma = pltpu.make_async_remote_copy(
            src_ref=comm_ref.at[6 + t],
            dst_ref=comm_ref.at[7 + t],
            send_sem=send_sems.at[3 + t],
            recv_sem=recv_sems.at[3 + t],
            device_id=(right,),
            device_id_type=pl.DeviceIdType.MESH,
        )
        rdma.start()
        rdma.wait()
        out_ref[chunk_rows(gr), :] = comm_ref[7 + t].astype(jnp.float32)


def kernel(x, Wq, K_ext, V_ext, Wo):
    my_pos = lax.axis_index("i")
    wq_loc = lax.dynamic_slice(
        Wq, (0, my_pos * D_LOC), (Wq.shape[0], D_LOC)
    ).astype(jnp.bfloat16)
    wq_loc = wq_loc.reshape(Wq.shape[0], H_LOC, DH).transpose(1, 0, 2)
    wo_loc = lax.dynamic_slice(
        Wo, (my_pos * D_LOC, 0), (D_LOC, Wo.shape[1])
    ).astype(jnp.bfloat16)
    wo_loc = wo_loc.reshape(H_LOC, DH, Wo.shape[1])
    xs = x[0].astype(jnp.bfloat16)
    k = K_ext[0].transpose(1, 0, 2).astype(jnp.bfloat16)
    v = V_ext[0].transpose(1, 0, 2).astype(jnp.bfloat16)

    out = pl.pallas_call(
        _body,
        out_shape=jax.ShapeDtypeStruct((SQ, D_MODEL), jnp.float32),
        in_specs=[pl.BlockSpec(memory_space=pltpu.VMEM)] * 5,
        out_specs=pl.BlockSpec(memory_space=pltpu.VMEM),
        scratch_shapes=[
            pltpu.VMEM((SQ, DH), jnp.bfloat16),
            pltpu.VMEM((SQ, DH), jnp.bfloat16),
            pltpu.VMEM((10, CHUNK, D_MODEL), jnp.bfloat16),
            pltpu.SemaphoreType.DMA((6,)),
            pltpu.SemaphoreType.DMA((6,)),
        ],
        compiler_params=pltpu.CompilerParams(
            collective_id=0,
            vmem_limit_bytes=60 * 1024 * 1024,
        ),
    )(xs, wq_loc, k, v, wo_loc)
    return out.reshape(1, SQ, D_MODEL)


# device time: 210360 ns/iter; 1.2483x vs baseline; 1.2483x over previous
import jax
import jax.numpy as jnp
from jax import lax
from jax.experimental import pallas as pl
from jax.experimental.pallas import tpu as pltpu

N_DEV = 4
SQ = 2048
SKV = 2048
H_LOC = 8
DH = 128
D_MODEL = 1024
D_LOC = H_LOC * DH
SCALE = 0.08838834764831843
Q_TILE = 512
N_QT = SQ // Q_TILE
CHUNK = SQ // N_DEV


def _body(x_ref, wq_ref, k_ref, v_ref, wo_ref, out_ref,
          qh_ref, ctx_ref, bias_ref, comm_ref, send_sems, recv_sems):
    my_pos = lax.axis_index("i")
    left = (my_pos - 1) % N_DEV
    right = (my_pos + 1) % N_DEV

    barrier_sem = pltpu.get_barrier_semaphore()
    for nbr in [left, right]:
        pl.semaphore_signal(
            barrier_sem, inc=1,
            device_id=(nbr,), device_id_type=pl.DeviceIdType.MESH,
        )
    pl.semaphore_wait(barrier_sem, 2)

    out_ref[...] = jnp.zeros((SQ, D_MODEL), jnp.float32)

    for t in range(N_QT):
        r0 = t * Q_TILE
        qb = (r0 + lax.broadcasted_iota(jnp.int32, (Q_TILE, SKV), 0)) // 64
        kb = lax.broadcasted_iota(jnp.int32, (Q_TILE, SKV), 1) // 64
        mask = (qb == kb) | (kb == 0) | ((qb + kb) % 3 == 0)
        bias_ref[r0:r0 + Q_TILE, :] = jnp.where(
            mask, 0.0, -1e9
        ).astype(jnp.bfloat16)

    def head_step(h, _):
        qh_ref[...] = jnp.dot(
            x_ref[...], wq_ref[h], preferred_element_type=jnp.float32
        ).astype(jnp.bfloat16)
        kh = k_ref[h]
        vh = v_ref[h]
        for t in range(N_QT):
            r0 = t * Q_TILE
            scores = lax.dot_general(
                qh_ref[r0:r0 + Q_TILE, :], kh, (((1,), (1,)), ((), ())),
                preferred_element_type=jnp.float32,
            ) + bias_ref[r0:r0 + Q_TILE, :].astype(jnp.float32)
            e = jnp.exp(scores)
            s = jnp.sum(e, axis=1, keepdims=True)
            w = (e * (1.0 / s)).astype(jnp.bfloat16)
            ctx_ref[r0:r0 + Q_TILE, :] = jnp.dot(
                w, vh, preferred_element_type=jnp.float32
            ).astype(jnp.bfloat16)
        out_ref[...] += jnp.dot(
            ctx_ref[...], wo_ref[h], preferred_element_type=jnp.float32
        )
        return 0

    lax.fori_loop(0, H_LOC, head_step, 0)


    def chunk_rows(c):
        return pl.ds(c * CHUNK, CHUNK)

    for s in range(N_DEV - 1):
        cs = (my_pos - s) % N_DEV
        cr = (my_pos - s - 1) % N_DEV
        comm_ref[s] = out_ref[chunk_rows(cs), :].astype(jnp.bfloat16)
        rdma = pltpu.make_async_remote_copy(
            src_ref=comm_ref.at[s],
            dst_ref=comm_ref.at[3 + s],
            send_sem=send_sems.at[s],
            recv_sem=recv_sems.at[s],
            device_id=(right,),
            device_id_type=pl.DeviceIdType.MESH,
        )
        rdma.start()
        rdma.wait()
        out_ref[chunk_rows(cr), :] += comm_ref[3 + s].astype(jnp.float32)

    comm_ref[6] = out_ref[chunk_rows((my_pos + 1) % N_DEV), :].astype(
        jnp.bfloat16
    )
    for t in range(N_DEV - 1):
        gr = (my_pos - t) % N_DEV
        rdma = pltpu.make_async_remote_copy(
            src_ref=comm_ref.at[6 + t],
            dst_ref=comm_ref.at[7 + t],
            send_sem=send_sems.at[3 + t],
            recv_sem=recv_sems.at[3 + t],
            device_id=(right,),
            device_id_type=pl.DeviceIdType.MESH,
        )
        rdma.start()
        rdma.wait()
        out_ref[chunk_rows(gr), :] = comm_ref[7 + t].astype(jnp.float32)


def kernel(x, Wq, K_ext, V_ext, Wo):
    my_pos = lax.axis_index("i")
    wq_loc = (
        lax.dynamic_slice(Wq, (0, my_pos * D_LOC), (Wq.shape[0], D_LOC))
        * SCALE
    ).astype(jnp.bfloat16)
    wq_loc = wq_loc.reshape(Wq.shape[0], H_LOC, DH).transpose(1, 0, 2)
    wo_loc = lax.dynamic_slice(
        Wo, (my_pos * D_LOC, 0), (D_LOC, Wo.shape[1])
    ).astype(jnp.bfloat16)
    wo_loc = wo_loc.reshape(H_LOC, DH, Wo.shape[1])
    xs = x[0].astype(jnp.bfloat16)
    k = K_ext[0].transpose(1, 0, 2).astype(jnp.bfloat16)
    v = V_ext[0].transpose(1, 0, 2).astype(jnp.bfloat16)

    out = pl.pallas_call(
        _body,
        out_shape=jax.ShapeDtypeStruct((SQ, D_MODEL), jnp.float32),
        in_specs=[pl.BlockSpec(memory_space=pltpu.VMEM)] * 5,
        out_specs=pl.BlockSpec(memory_space=pltpu.VMEM),
        scratch_shapes=[
            pltpu.VMEM((SQ, DH), jnp.bfloat16),
            pltpu.VMEM((SQ, DH), jnp.bfloat16),
            pltpu.VMEM((SQ, SKV), jnp.bfloat16),
            pltpu.VMEM((10, CHUNK, D_MODEL), jnp.bfloat16),
            pltpu.SemaphoreType.DMA((6,)),
            pltpu.SemaphoreType.DMA((6,)),
        ],
        compiler_params=pltpu.CompilerParams(
            collective_id=0,
            vmem_limit_bytes=60 * 1024 * 1024,
        ),
    )(xs, wq_loc, k, v, wo_loc)
    return out.reshape(1, SQ, D_MODEL)


# device time: 164361 ns/iter; 1.5976x vs baseline; 1.2799x over previous
import jax
import jax.numpy as jnp
from jax import lax
from jax.experimental import pallas as pl
from jax.experimental.pallas import tpu as pltpu

N_DEV = 4
SQ = 2048
SKV = 2048
H_LOC = 8
DH = 128
D_MODEL = 1024
D_LOC = H_LOC * DH
SCALE = 0.08838834764831843
CHUNK = SQ // N_DEV


def _body(x_ref, wq_ref, k_ref, v_ref, wo_ref, out_ref,
          ctx_ref, bias_ref, comm_ref, send_sems, recv_sems):
    my_pos = lax.axis_index("i")
    left = (my_pos - 1) % N_DEV
    right = (my_pos + 1) % N_DEV

    barrier_sem = pltpu.get_barrier_semaphore()
    for nbr in [left, right]:
        pl.semaphore_signal(
            barrier_sem, inc=1,
            device_id=(nbr,), device_id_type=pl.DeviceIdType.MESH,
        )
    pl.semaphore_wait(barrier_sem, 2)

    for t in range(N_DEV):
        r0 = t * CHUNK
        qb = (r0 + lax.broadcasted_iota(jnp.int32, (CHUNK, SKV), 0)) // 64
        kb = lax.broadcasted_iota(jnp.int32, (CHUNK, SKV), 1) // 64
        mask = (qb == kb) | (kb == 0) | ((qb + kb) % 3 == 0)
        bias_ref[r0:r0 + CHUNK, :] = jnp.where(
            mask, 0.0, -1e9
        ).astype(jnp.bfloat16)

    def rdma(slot_s, slot_d, sem):
        return pltpu.make_async_remote_copy(
            src_ref=comm_ref.at[slot_s],
            dst_ref=comm_ref.at[slot_d],
            send_sem=send_sems.at[sem],
            recv_sem=recv_sems.at[sem],
            device_id=(right,),
            device_id_type=pl.DeviceIdType.MESH,
        )

    def partial_chunk(c):
        rows = pl.ds(c * CHUNK, CHUNK)
        xr = x_ref[rows, :]
        br = bias_ref[rows, :].astype(jnp.float32)
        for h in range(H_LOC):
            qh = jnp.dot(
                xr, wq_ref[h], preferred_element_type=jnp.float32
            ).astype(jnp.bfloat16)
            scores = lax.dot_general(
                qh, k_ref[h], (((1,), (1,)), ((), ())),
                preferred_element_type=jnp.float32,
            ) + br
            e = jnp.exp(scores)
            s = jnp.sum(e, axis=1, keepdims=True)
            w = (e * (1.0 / s)).astype(jnp.bfloat16)
            ctx_ref[:, h * DH:(h + 1) * DH] = jnp.dot(
                w, v_ref[h], preferred_element_type=jnp.float32
            ).astype(jnp.bfloat16)
        return jnp.dot(
            ctx_ref[...], wo_ref[...], preferred_element_type=jnp.float32
        )

    sends = []
    p = partial_chunk(my_pos % N_DEV)
    comm_ref[0] = p.astype(jnp.bfloat16)
    r = rdma(0, 3, 0)
    r.start()
    sends.append(r)
    for j in range(1, N_DEV):
        p = partial_chunk((my_pos - j) % N_DEV)
        sends[j - 1].wait_recv()
        p = p + comm_ref[2 + j].astype(jnp.float32)
        if j < N_DEV - 1:
            comm_ref[j] = p.astype(jnp.bfloat16)
            r = rdma(j, 3 + j, j)
            r.start()
            sends.append(r)
    out_ref[pl.ds(((my_pos + 1) % N_DEV) * CHUNK, CHUNK), :] = p
    comm_ref[6] = p.astype(jnp.bfloat16)

    r = rdma(6, 7, 3)
    r.start()
    sends.append(r)
    for t in range(N_DEV - 1):
        sends[3 + t].wait_recv()
        if t < N_DEV - 2:
            r = rdma(7 + t, 8 + t, 4 + t)
            r.start()
            sends.append(r)
        gr = (my_pos - t) % N_DEV
        out_ref[pl.ds(gr * CHUNK, CHUNK), :] = comm_ref[7 + t].astype(
            jnp.float32
        )
    for r in sends:
        r.wait_send()


def kernel(x, Wq, K_ext, V_ext, Wo):
    my_pos = lax.axis_index("i")
    wq_loc = (
        lax.dynamic_slice(Wq, (0, my_pos * D_LOC), (Wq.shape[0], D_LOC))
        * SCALE
    ).astype(jnp.bfloat16)
    wq_loc = wq_loc.reshape(Wq.shape[0], H_LOC, DH).transpose(1, 0, 2)
    wo_loc = lax.dynamic_slice(
        Wo, (my_pos * D_LOC, 0), (D_LOC, Wo.shape[1])
    ).astype(jnp.bfloat16)
    xs = x[0].astype(jnp.bfloat16)
    k = K_ext[0].transpose(1, 0, 2).astype(jnp.bfloat16)
    v = V_ext[0].transpose(1, 0, 2).astype(jnp.bfloat16)

    out = pl.pallas_call(
        _body,
        out_shape=jax.ShapeDtypeStruct((SQ, D_MODEL), jnp.float32),
        in_specs=[pl.BlockSpec(memory_space=pltpu.VMEM)] * 5,
        out_specs=pl.BlockSpec(memory_space=pltpu.VMEM),
        scratch_shapes=[
            pltpu.VMEM((CHUNK, D_LOC), jnp.bfloat16),
            pltpu.VMEM((SQ, SKV), jnp.bfloat16),
            pltpu.VMEM((10, CHUNK, D_MODEL), jnp.bfloat16),
            pltpu.SemaphoreType.DMA((6,)),
            pltpu.SemaphoreType.DMA((6,)),
        ],
        compiler_params=pltpu.CompilerParams(
            collective_id=0,
            vmem_limit_bytes=62 * 1024 * 1024,
        ),
    )(xs, wq_loc, k, v, wo_loc)
    return out.reshape(1, SQ, D_MODEL)
